# Initial kernel scaffold; baseline (speedup 1.0000x reference)
import jax
import jax.numpy as jnp
from jax.experimental import pallas as pl


def kernel(x, table_0, table_1, table_2, table_3, table_4, table_5, table_6, table_7, table_8, table_9, table_10, table_11, table_12, table_13, table_14, table_15):
    raise NotImplementedError("write your pallas kernel here")


# speedup vs baseline: 35.1800x; 35.1800x over previous
import math

import numpy as np
import jax
import jax.numpy as jnp
from jax import lax
from jax.experimental import pallas as pl
from jax.experimental.pallas import tpu as pltpu
from jax.experimental.pallas import tpu_sc as plsc

IMG_H = 2048
IMG_W = 2048
N_LEVELS = 16
LOG2T = 19
T = 1 << LOG2T
BASE = 16
FINEST = 2048
N_POINTS = 262144

_B = np.exp((np.log(np.float32(FINEST)) - np.log(np.float32(BASE)))
            / np.float32(N_LEVELS - 1)).astype(np.float32)
RES = [int(math.floor(float(np.float32(BASE) * (_B ** np.float32(i)))))
       for i in range(N_LEVELS)]
TABLE_SIZES = [(r + 1) ** 2 if r * r < T else T for r in RES]
FINE_LEVELS = [l for l in range(N_LEVELS) if (RES[l] + 1) ** 2 > T]
IDX2RAD = 2.0 * math.pi / T
HASH_MULT = np.int32(np.uint32(2654435761).astype(np.int64) - (1 << 32))
GS = [np.float32(IMG_H / r) for r in RES]


def _build_remaps():
    remaps = []
    for lvl in FINE_LEVELS:
        r = RES[lvl]
        a = np.arange(r + 1)
        xg, yg = np.meshgrid(a, a, indexing='xy')
        xy = np.stack([xg, yg], axis=-1).reshape(-1, 2)
        h = ((xy[:, 0].astype(np.uint32) * np.uint32(1))
             ^ (xy[:, 1].astype(np.uint32) * np.uint32(2654435761)))
        h = (h & np.uint32(T - 1)).astype(np.int64)
        rad = h.astype(np.float32) * np.float32(IDX2RAD)
        order = np.argsort(rad, kind='stable')
        rad_s = rad[order]
        h_s = h[order]
        tkey = (rad_s + np.concatenate(
            [rad_s[1:], np.array([2.0 * np.pi], dtype=np.float32)])
        ) / np.float32(2.0)
        q = np.arange(T, dtype=np.int64).astype(np.float32) * np.float32(IDX2RAD)
        pos = np.searchsorted(tkey, q, side='left') % tkey.shape[0]
        remaps.append(h_s[pos].astype(np.int32))
    return remaps


REMAPS = _build_remaps()

NW = 32
NPW = N_POINTS // NW
C = 256
NCH = NPW // C
NCHUNK_TOT = N_POINTS // C
K = 4 * C


def _body(*refs):
    it = iter(refs)
    x0_hbm = next(it)
    x1_hbm = next(it)
    tabsA = [next(it) for _ in range(N_LEVELS)]
    tabsB = [next(it) for _ in range(N_LEVELS)]
    rems = [next(it) for _ in range(len(FINE_LEVELS))]
    out_hbm = next(it)
    hids = [next(it) for _ in range(N_LEVELS)]
    x0b = next(it)
    x1b = next(it)
    outb = next(it)
    hbs = [next(it) for _ in range(N_LEVELS)]
    hss = [next(it) for _ in range(len(FINE_LEVELS))]
    ras = [next(it) for _ in range(N_LEVELS)]
    rbs = [next(it) for _ in range(N_LEVELS)]
    gsem = next(it)
    osem = next(it)

    cid = lax.axis_index("c")
    sid = lax.axis_index("s")
    wid = sid * 2 + cid

    lanes = lax.iota(jnp.int32, 16)

    def chunk_body(g, carry):
        chunk = wid * NCH + g
        pltpu.sync_copy(x0_hbm.at[chunk], x0b)
        pltpu.sync_copy(x1_hbm.at[chunk], x1b)

        def p1(v, c2):
            s = pl.multiple_of(v * 16, 16)
            x0 = x0b[pl.ds(s, 16)]
            x1 = x1b[pl.ds(s, 16)]
            rowb = (lanes + s) * 4
            for l in range(N_LEVELS):
                res = RES[l]
                b0 = (x0 / GS[l]).astype(jnp.int32)
                b1 = (x1 / GS[l]).astype(jnp.int32)
                if l not in FINE_LEVELS:
                    base_i = b0 * res + b1
                    v00 = base_i
                    v01 = base_i + 1
                    v10 = base_i + res
                    v11 = base_i + res + 1
                    dst = hbs[l]
                else:
                    m1 = b1 * HASH_MULT
                    m1b = (b1 + 1) * HASH_MULT
                    v00 = (b0 ^ m1) & (T - 1)
                    v01 = (b0 ^ m1b) & (T - 1)
                    v10 = ((b0 + 1) ^ m1) & (T - 1)
                    v11 = ((b0 + 1) ^ m1b) & (T - 1)
                    dst = hss[FINE_LEVELS.index(l)]
                plsc.store_scatter(dst, [rowb], v00)
                plsc.store_scatter(dst, [rowb + 1], v01)
                plsc.store_scatter(dst, [rowb + 2], v10)
                plsc.store_scatter(dst, [rowb + 3], v11)
            return c2

        lax.fori_loop(0, C // 16, p1, 0)

        rcopies = [pltpu.async_copy(rems[k].at[hss[k]], hbs[l], gsem)
                   for k, l in enumerate(FINE_LEVELS)]
        gcopies = []
        for l in range(N_LEVELS):
            if l not in FINE_LEVELS:
                gcopies.append(pltpu.async_copy(tabsA[l].at[hbs[l]], ras[l], gsem))
                gcopies.append(pltpu.async_copy(tabsB[l].at[hbs[l]], rbs[l], gsem))
        for cp in rcopies:
            cp.wait()
        for l in FINE_LEVELS:
            gcopies.append(pltpu.async_copy(tabsA[l].at[hbs[l]], ras[l], gsem))
            gcopies.append(pltpu.async_copy(tabsB[l].at[hbs[l]], rbs[l], gsem))
        for cp in gcopies:
            cp.wait()

        def p2(v, c2):
            s = pl.multiple_of(v * 16, 16)
            x0 = x0b[pl.ds(s, 16)]
            x1 = x1b[pl.ds(s, 16)]
            rowb = (lanes + s) * 4
            ob = (lanes + s) * 32
            for l in range(N_LEVELS):
                gs = GS[l]
                b0f = (x0 / gs).astype(jnp.int32).astype(jnp.float32)
                b1f = (x1 / gs).astype(jnp.int32).astype(jnp.float32)
                g0 = b0f * gs
                g1 = b1f * gs
                d0 = (g0 + gs) - g0
                d1 = (g1 + gs) - g1
                w0 = (x0 - g0) / d0
                w1 = (x1 - g1) / d1
                e0a = plsc.load_gather(ras[l], [rowb])
                e0b = plsc.load_gather(rbs[l], [rowb])
                e1a = plsc.load_gather(ras[l], [rowb + 1])
                e1b = plsc.load_gather(rbs[l], [rowb + 1])
                e2a = plsc.load_gather(ras[l], [rowb + 2])
                e2b = plsc.load_gather(rbs[l], [rowb + 2])
                e3a = plsc.load_gather(ras[l], [rowb + 3])
                e3b = plsc.load_gather(rbs[l], [rowb + 3])
                t1 = 1.0 - w1
                t0 = 1.0 - w0
                ca = (e0a * t1 + e1a * w1) * t0 + (e2a * t1 + e3a * w1) * w0
                cb = (e0b * t1 + e1b * w1) * t0 + (e2b * t1 + e3b * w1) * w0
                plsc.store_scatter(outb, [ob + (2 * l)], ca)
                plsc.store_scatter(outb, [ob + (2 * l + 1)], cb)
            return c2

        lax.fori_loop(0, C // 16, p2, 0)

        ocs = [pltpu.async_copy(outb, out_hbm.at[chunk], osem)]
        for l in range(N_LEVELS):
            ocs.append(pltpu.async_copy(hbs[l], hids[l].at[chunk], osem))
        for cp in ocs:
            cp.wait()
        return carry

    lax.fori_loop(0, NCH, chunk_body, 0)


_OUT_TYPE = ([jax.ShapeDtypeStruct((NCHUNK_TOT, C * 2 * N_LEVELS), jnp.float32)]
             + [jax.ShapeDtypeStruct((NCHUNK_TOT, K), jnp.int32)
                for _ in range(N_LEVELS)])

_SCRATCH = ([pltpu.VMEM((C,), jnp.float32),
             pltpu.VMEM((C,), jnp.float32),
             pltpu.VMEM((C * 2 * N_LEVELS,), jnp.float32)]
            + [pltpu.VMEM((K,), jnp.int32) for _ in range(N_LEVELS)]
            + [pltpu.VMEM((K,), jnp.int32) for _ in range(len(FINE_LEVELS))]
            + [pltpu.VMEM((K,), jnp.float32) for _ in range(N_LEVELS)]
            + [pltpu.VMEM((K,), jnp.float32) for _ in range(N_LEVELS)]
            + [pltpu.SemaphoreType.DMA, pltpu.SemaphoreType.DMA])


def _make_kfn():
    mesh = plsc.VectorSubcoreMesh(core_axis_name="c", subcore_axis_name="s",
                                  num_cores=2, num_subcores=16)
    return pl.kernel(
        _body, out_type=_OUT_TYPE, mesh=mesh, scratch_types=_SCRATCH,
        compiler_params=pltpu.CompilerParams(needs_layout_passes=False))


@jax.jit
def _run(x, *tables):
    x0 = x[:, 0].reshape(NCHUNK_TOT, C)
    x1 = x[:, 1].reshape(NCHUNK_TOT, C)
    tabsA = [t[:, 0] for t in tables]
    tabsB = [t[:, 1] for t in tables]
    rems = [jnp.asarray(r) for r in REMAPS]
    outs = _make_kfn()(x0, x1, *tabsA, *tabsB, *rems)
    out = outs[0].reshape(N_POINTS, 2 * N_LEVELS)
    hid = [o.reshape(N_POINTS, 4) for o in outs[1:]]
    return (out,) + tuple(hid)


def kernel(x, table_0, table_1, table_2, table_3, table_4, table_5, table_6,
           table_7, table_8, table_9, table_10, table_11, table_12, table_13,
           table_14, table_15):
    return _run(x, table_0, table_1, table_2, table_3, table_4, table_5,
                table_6, table_7, table_8, table_9, table_10, table_11,
                table_12, table_13, table_14, table_15)
